# baseline (speedup 1.0000x reference)
import functools

import jax
import jax.numpy as jnp
from jax import lax
from jax.experimental import pallas as pl
from jax.experimental.pallas import tpu as pltpu
from jax.experimental.pallas import tpu_sc as plsc

B = 4
S = 8192
D = 768
NC = 2
NS = 16
NW = NC * NS
S_W = S // NW
R = 16
NBUF = 2
NCHUNK = S_W // R
NGRP = NCHUNK // NBUF
NV = D // 16

_mesh = plsc.VectorSubcoreMesh(core_axis_name="c", subcore_axis_name="s")


@functools.partial(
    pl.kernel,
    mesh=_mesh,
    out_type=jax.ShapeDtypeStruct((B, S, D), jnp.float32),
    scratch_types=(
        [pltpu.VMEM((S_W,), jnp.int32), pltpu.VMEM((16,), jnp.int32)]
        + [pltpu.VMEM((R, D), jnp.float32) for _ in range(NBUF)]
        + [pltpu.VMEM((B, R, D), jnp.float32) for _ in range(NBUF)]
        + [pltpu.SemaphoreType.DMA for _ in range(2 * NBUF)]
    ),
)
def _pe_add(enc_hbm, t_hbm, table_hbm, out_hbm, idx_v, t0_v, *bufs):
    rows = bufs[0:NBUF]
    encb = bufs[NBUF:2 * NBUF]
    sem_in = bufs[2 * NBUF:3 * NBUF]
    sem_st = bufs[3 * NBUF:4 * NBUF]

    cid = lax.axis_index("c")
    sid = lax.axis_index("s")
    wid = sid * NC + cid
    base = wid * S_W
    def issue_enc_in(i, buf):
        row0 = i * R
        pltpu.async_copy(
            enc_hbm.at[:, pl.ds(base + row0, R), :], encb[buf], sem_in[buf]
        )

    def issue_gather(i, buf):
        row0 = i * R
        pltpu.async_copy(
            table_hbm.at[idx_v.at[pl.ds(row0, R)]], rows[buf], sem_in[buf]
        )

    for p in range(NBUF - 1):
        issue_enc_in(p, p)

    pltpu.sync_copy(t_hbm.at[pl.ds(base, S_W)], idx_v)
    pltpu.sync_copy(t_hbm.at[pl.ds(0, 16)], t0_v)
    z16 = lax.iota(jnp.int32, 16)
    t0 = t0_v[...].at[z16 - z16].get(mode="promise_in_bounds")
    for q in range(S_W // 16):
        sl = pl.ds(q * 16, 16)
        idx_v[sl] = idx_v[sl] - t0

    def issue_in(i, buf):
        row0 = i * R
        pltpu.async_copy(
            table_hbm.at[idx_v.at[pl.ds(row0, R)]], rows[buf], sem_in[buf]
        )
        pltpu.async_copy(
            enc_hbm.at[:, pl.ds(base + row0, R), :], encb[buf], sem_in[buf]
        )

    def wait_in(buf):
        pltpu.make_async_copy(
            table_hbm.at[pl.ds(0, R)], rows[buf], sem_in[buf]
        ).wait()
        pltpu.make_async_copy(
            enc_hbm.at[:, pl.ds(0, R), :], encb[buf], sem_in[buf]
        ).wait()

    def wait_store(buf):
        pltpu.make_async_copy(
            encb[buf], out_hbm.at[:, pl.ds(0, R), :], sem_st[buf]
        ).wait()

    def compute(buf):
        @plsc.parallel_loop(0, R)
        def _rbody(r):
            row_next = rows[buf][r, pl.ds(0, 16)]
            for j in range(NV):
                sl = pl.ds(j * 16, 16)
                row = row_next
                if j + 1 < NV:
                    row_next = rows[buf][r, pl.ds((j + 1) * 16, 16)]
                for bb in range(B):
                    plsc.addupdate(encb[buf].at[bb, r, sl], row)

    for p in range(NBUF - 1):
        issue_gather(p, p)

    def grp(g, _):
        for b in range(NBUF):
            i = g * NBUF + b
            wait_in(b)
            pb = (b + NBUF - 1) % NBUF

            @pl.when(i >= 1)
            def _():
                wait_store(pb)

            @pl.when(i + NBUF - 1 < NCHUNK)
            def _():
                issue_in(i + NBUF - 1, pb)

            compute(b)
            row0 = i * R
            pltpu.async_copy(
                encb[b], out_hbm.at[:, pl.ds(base + row0, R), :], sem_st[b]
            )

        return 0

    lax.fori_loop(0, NGRP, grp, 0)
    wait_store((NCHUNK - 1) % NBUF)


def kernel(enc_inputs, tindex, pos_table):
    return _pe_add(enc_inputs, tindex, pos_table)
